# baseline (speedup 1.0000x reference)
import functools

import jax
import jax.numpy as jnp
from jax import lax
from jax.experimental import pallas as pl
from jax.experimental.pallas import tpu as pltpu
from jax.experimental.pallas import tpu_sc as plsc

D = 32
H = 128
T = 9
L_SEQ = 50
P = 4

_NC = 2
_NS = 16
_NW = _NC * _NS
_CHUNK = 128


def _sc_gather(table, idx3, n_rows):
    n_chunks = idx3.shape[1]
    mesh = plsc.VectorSubcoreMesh(core_axis_name="c", subcore_axis_name="s")

    @functools.partial(
        pl.kernel,
        mesh=mesh,
        out_type=jax.ShapeDtypeStruct((n_rows, D), jnp.float32),
        compiler_params=pltpu.CompilerParams(use_tc_tiling_on_sc=False),
        scratch_types=[
            pltpu.VMEM((n_chunks, _CHUNK), jnp.int32),
            pltpu.VMEM((_CHUNK, D), jnp.float32),
            pltpu.SemaphoreType.DMA,
        ],
    )
    def k(table_hbm, idx_hbm, out_hbm, idx_v, rows_v, sem):
        wid = lax.axis_index("s") * _NC + lax.axis_index("c")
        pltpu.sync_copy(idx_hbm.at[wid], idx_v)

        def body(j, carry):
            pltpu.async_copy(table_hbm.at[idx_v.at[j]], rows_v, sem).wait()
            base = (wid * n_chunks + j) * _CHUNK
            pltpu.sync_copy(rows_v, out_hbm.at[pl.ds(base, _CHUNK)])
            return carry

        lax.fori_loop(0, n_chunks, body, 0)

    return k(table, idx3)


def _sigmoid(x):
    return 0.5 * jnp.tanh(0.5 * x) + 0.5


def _lstm_body(x_ref, wih_ref, whh_ref, b_ref, wfc_ref, bfc_ref, out_ref,
               hs_ref):
    wih = wih_ref[...]
    whh = whh_ref[...]
    b = b_ref[...]
    bp = x_ref.shape[1]
    HP = P * H

    def step(t, carry):
        h, c = carry
        x_t = x_ref[t]
        gates = (jnp.dot(x_t, wih, preferred_element_type=jnp.float32)
                 + jnp.dot(h, whh, preferred_element_type=jnp.float32) + b)
        ig = _sigmoid(gates[:, 0:HP])
        fg = _sigmoid(gates[:, HP:2 * HP])
        gg = jnp.tanh(gates[:, 2 * HP:3 * HP])
        og = _sigmoid(gates[:, 3 * HP:4 * HP])
        c = fg * c + ig * gg
        h = og * jnp.tanh(c)
        hs_ref[pl.ds(t * bp, bp), :] = h
        return (h, c)

    init = (jnp.zeros((bp, HP), jnp.float32), jnp.zeros((bp, HP), jnp.float32))
    lax.fori_loop(0, L_SEQ, step, init)

    wfc9 = wfc_ref[...]
    bfc9 = bfc_ref[...]
    for s in range(P):
        h_s = hs_ref[:, s * H:(s + 1) * H]
        lT = jax.lax.dot_general(wfc9, h_s, (((1,), (1,)), ((), ())),
                                 preferred_element_type=jnp.float32) + bfc9
        m = jnp.max(lT, axis=0, keepdims=True)
        e = jnp.exp(lT - m)
        lse = m + jnp.log(jnp.sum(e, axis=0, keepdims=True))
        out_ref[:, s, :, :] = (lT - lse).reshape(T, L_SEQ, bp)


def _lstm_fc(x, wih, whh, bias, wfc, bfc, bp=128):
    BP = x.shape[1]

    return pl.pallas_call(
        _lstm_body,
        grid=(BP // bp,),
        in_specs=[
            pl.BlockSpec((L_SEQ, bp, P * D), lambda i: (0, i, 0)),
            pl.BlockSpec((P * D, P * 4 * H), lambda i: (0, 0)),
            pl.BlockSpec((P * H, P * 4 * H), lambda i: (0, 0)),
            pl.BlockSpec((1, P * 4 * H), lambda i: (0, 0)),
            pl.BlockSpec((T, H), lambda i: (0, 0)),
            pl.BlockSpec((T, 1), lambda i: (0, 0)),
        ],
        out_specs=pl.BlockSpec((T, P, L_SEQ, bp), lambda i: (0, 0, 0, i)),
        out_shape=jax.ShapeDtypeStruct((T, P, L_SEQ, BP), jnp.float32),
        scratch_shapes=[pltpu.VMEM((L_SEQ * bp, P * H), jnp.float32)],
    )(x, wih, whh, bias, wfc, bfc)


def _pack_w(w, eye):
    k = w.shape[0]
    wg = w.reshape(k, 4, H)
    m = jnp.einsum('st,kgh->skgth', eye, wg)
    return m.reshape(P * k, P * 4 * H)


def kernel(sentences, labels, emb_table, W_ih, W_hh, b_ih, b_hh, W_fc, b_fc):
    B, L = sentences.shape
    n_rows = B * L
    BP = B // P
    sp = sentences.reshape(P, BP, L)
    idx = jnp.transpose(sp, (2, 1, 0)).reshape(-1).astype(jnp.int32)
    idx3 = idx.reshape(_NW, -1, _CHUNK)
    x = _sc_gather(emb_table, idx3, n_rows).reshape(L, BP, P * D)

    eye = jnp.eye(P, dtype=jnp.float32)
    wih = _pack_w(W_ih.T, eye)
    whh = _pack_w(W_hh.T, eye)
    bg = (b_ih + b_hh).reshape(4, H)
    bias = jnp.broadcast_to(bg[:, None, :], (4, P, H)).reshape(1, P * 4 * H)
    bfc = b_fc.reshape(T, 1)

    out4 = _lstm_fc(x, wih, whh, bias, W_fc, bfc)
    return jnp.transpose(out4, (1, 3, 2, 0)).reshape(B, L, T)


# speedup vs baseline: 1.0213x; 1.0213x over previous
import functools

import jax
import jax.numpy as jnp
from jax import lax
from jax.experimental import pallas as pl
from jax.experimental.pallas import tpu as pltpu
from jax.experimental.pallas import tpu_sc as plsc

D = 32
H = 128
T = 9
L_SEQ = 50
P = 4

_NC = 2
_NS = 16
_NW = _NC * _NS
_CHUNK = 128


def _sc_gather(table, idx3, n_rows):
    n_chunks = idx3.shape[1]
    mesh = plsc.VectorSubcoreMesh(core_axis_name="c", subcore_axis_name="s")

    @functools.partial(
        pl.kernel,
        mesh=mesh,
        out_type=jax.ShapeDtypeStruct((n_rows, D), jnp.float32),
        compiler_params=pltpu.CompilerParams(use_tc_tiling_on_sc=False),
        scratch_types=[
            pltpu.VMEM((n_chunks, _CHUNK), jnp.int32),
            pltpu.VMEM((_CHUNK, D), jnp.float32),
            pltpu.SemaphoreType.DMA,
        ],
    )
    def k(table_hbm, idx_hbm, out_hbm, idx_v, rows_v, sem):
        wid = lax.axis_index("s") * _NC + lax.axis_index("c")
        pltpu.sync_copy(idx_hbm.at[wid], idx_v)

        def body(j, carry):
            pltpu.async_copy(table_hbm.at[idx_v.at[j]], rows_v, sem).wait()
            base = (wid * n_chunks + j) * _CHUNK
            pltpu.sync_copy(rows_v, out_hbm.at[pl.ds(base, _CHUNK)])
            return carry

        lax.fori_loop(0, n_chunks, body, 0)

    return k(table, idx3)


def _sigmoid(x):
    return 0.5 * jnp.tanh(0.5 * x) + 0.5


def _lstm_body(x_ref, wih_ref, whh_ref, b_ref, wfc_ref, bfc_ref, out_ref,
               hs_ref):
    wih = wih_ref[...]
    whh = whh_ref[...]
    b = b_ref[...]
    bp = x_ref.shape[1]
    HP = P * H

    def step(t, carry):
        h, c = carry
        x_t = x_ref[t]
        gates = (jnp.dot(x_t, wih, preferred_element_type=jnp.float32)
                 + jnp.dot(h, whh, preferred_element_type=jnp.float32) + b)
        ig = _sigmoid(gates[:, 0:HP])
        fg = _sigmoid(gates[:, HP:2 * HP])
        gg = jnp.tanh(gates[:, 2 * HP:3 * HP])
        og = _sigmoid(gates[:, 3 * HP:4 * HP])
        c = fg * c + ig * gg
        h = og * jnp.tanh(c)
        hs_ref[pl.ds(t * bp, bp), :] = h
        return (h, c)

    init = (jnp.zeros((bp, HP), jnp.float32), jnp.zeros((bp, HP), jnp.float32))
    lax.fori_loop(0, L_SEQ, step, init)

    wfc9 = wfc_ref[...]
    bfc9 = bfc_ref[...]
    for s in range(P):
        h_s = hs_ref[:, s * H:(s + 1) * H]
        lT = jax.lax.dot_general(wfc9, h_s, (((1,), (1,)), ((), ())),
                                 preferred_element_type=jnp.float32) + bfc9
        m = jnp.max(lT, axis=0, keepdims=True)
        e = jnp.exp(lT - m)
        lse = m + jnp.log(jnp.sum(e, axis=0, keepdims=True))
        out_ref[:, :, s * bp:(s + 1) * bp] = (lT - lse).reshape(T, L_SEQ, bp)


def _lstm_fc(x, wih, whh, bias, wfc, bfc, bp=128):
    BP = x.shape[1]

    return pl.pallas_call(
        _lstm_body,
        grid=(BP // bp,),
        in_specs=[
            pl.BlockSpec((L_SEQ, bp, P * D), lambda i: (0, i, 0)),
            pl.BlockSpec((P * D, P * 4 * H), lambda i: (0, 0)),
            pl.BlockSpec((P * H, P * 4 * H), lambda i: (0, 0)),
            pl.BlockSpec((1, P * 4 * H), lambda i: (0, 0)),
            pl.BlockSpec((T, H), lambda i: (0, 0)),
            pl.BlockSpec((T, 1), lambda i: (0, 0)),
        ],
        out_specs=pl.BlockSpec((T, L_SEQ, P * bp), lambda i: (0, 0, i)),
        out_shape=jax.ShapeDtypeStruct((T, L_SEQ, P * BP), jnp.float32),
        scratch_shapes=[pltpu.VMEM((L_SEQ * bp, P * H), jnp.float32)],
    )(x, wih, whh, bias, wfc, bfc)


def _pack_w(w, eye):
    k = w.shape[0]
    wg = w.reshape(k, 4, H)
    m = jnp.einsum('st,kgh->skgth', eye, wg)
    return m.reshape(P * k, P * 4 * H)


def kernel(sentences, labels, emb_table, W_ih, W_hh, b_ih, b_hh, W_fc, b_fc):
    B, L = sentences.shape
    n_rows = B * L
    BP = B // P
    bp = 128
    sp = sentences.reshape(B // (P * bp), P, bp, L)
    idx = jnp.transpose(sp, (3, 0, 2, 1)).reshape(-1).astype(jnp.int32)
    idx3 = idx.reshape(_NW, -1, _CHUNK)
    x = _sc_gather(emb_table, idx3, n_rows).reshape(L, BP, P * D)

    eye = jnp.eye(P, dtype=jnp.float32)
    wih = _pack_w(W_ih.T, eye)
    whh = _pack_w(W_hh.T, eye)
    bg = (b_ih + b_hh).reshape(4, H)
    bias = jnp.broadcast_to(bg[:, None, :], (4, P, H)).reshape(1, P * 4 * H)
    bfc = b_fc.reshape(T, 1)

    out_t = _lstm_fc(x, wih, whh, bias, W_fc, bfc, bp=bp)
    return jnp.transpose(out_t, (2, 1, 0))


# speedup vs baseline: 1.0361x; 1.0145x over previous
import functools

import jax
import jax.numpy as jnp
from jax import lax
from jax.experimental import pallas as pl
from jax.experimental.pallas import tpu as pltpu
from jax.experimental.pallas import tpu_sc as plsc

D = 32
H = 128
T = 9
L_SEQ = 50

_NC = 2
_NS = 16
_NW = _NC * _NS
_CHUNK = 128


def _sc_gather(table, idx3, n_rows):
    n_chunks = idx3.shape[1]
    mesh = plsc.VectorSubcoreMesh(core_axis_name="c", subcore_axis_name="s")

    @functools.partial(
        pl.kernel,
        mesh=mesh,
        out_type=jax.ShapeDtypeStruct((n_rows, D), jnp.float32),
        compiler_params=pltpu.CompilerParams(use_tc_tiling_on_sc=False),
        scratch_types=[
            pltpu.VMEM((n_chunks, _CHUNK), jnp.int32),
            pltpu.VMEM((_CHUNK, D), jnp.float32),
            pltpu.SemaphoreType.DMA,
        ],
    )
    def k(table_hbm, idx_hbm, out_hbm, idx_v, rows_v, sem):
        wid = lax.axis_index("s") * _NC + lax.axis_index("c")
        pltpu.sync_copy(idx_hbm.at[wid], idx_v)

        def body(j, carry):
            pltpu.async_copy(table_hbm.at[idx_v.at[j]], rows_v, sem).wait()
            base = (wid * n_chunks + j) * _CHUNK
            pltpu.sync_copy(rows_v, out_hbm.at[pl.ds(base, _CHUNK)])
            return carry

        lax.fori_loop(0, n_chunks, body, 0)

    return k(table, idx3)


def _sigmoid(x):
    return 0.5 * jnp.tanh(0.5 * x) + 0.5


def _lstm_body(x_ref, wih_ref, whh_ref, b_ref, wfc_ref, bfc_ref, out_ref,
               hs_ref):
    wih = wih_ref[...]
    whh = whh_ref[...]
    b = b_ref[...]
    bt = x_ref.shape[1]

    def step(t, carry):
        h, c = carry
        x_t = x_ref[t]
        gates = (jnp.dot(x_t, wih, preferred_element_type=jnp.float32)
                 + jnp.dot(h, whh, preferred_element_type=jnp.float32) + b)
        ig = _sigmoid(gates[:, 0:H])
        fg = _sigmoid(gates[:, H:2 * H])
        gg = jnp.tanh(gates[:, 2 * H:3 * H])
        og = _sigmoid(gates[:, 3 * H:4 * H])
        c = fg * c + ig * gg
        h = og * jnp.tanh(c)
        hs_ref[pl.ds(t * bt, bt), :] = h
        return (h, c)

    init = (jnp.zeros((bt, H), jnp.float32), jnp.zeros((bt, H), jnp.float32))
    lax.fori_loop(0, L_SEQ, step, init)

    wfc9 = wfc_ref[...]
    bfc9 = bfc_ref[...]
    for t in range(L_SEQ):
        h_t = hs_ref[pl.ds(t * bt, bt), :]
        lT = jax.lax.dot_general(wfc9, h_t, (((1,), (1,)), ((), ())),
                                 preferred_element_type=jnp.float32) + bfc9
        m = jnp.max(lT, axis=0, keepdims=True)
        e = jnp.exp(lT - m)
        lse = m + jnp.log(jnp.sum(e, axis=0, keepdims=True))
        out_ref[:, t, :] = lT - lse


def _lstm_fc(x, wih, whh, bias, wfc, bfc, bt=512):
    B = x.shape[1]
    return pl.pallas_call(
        _lstm_body,
        grid=(B // bt,),
        in_specs=[
            pl.BlockSpec((L_SEQ, bt, D), lambda i: (0, i, 0)),
            pl.BlockSpec((D, 4 * H), lambda i: (0, 0)),
            pl.BlockSpec((H, 4 * H), lambda i: (0, 0)),
            pl.BlockSpec((1, 4 * H), lambda i: (0, 0)),
            pl.BlockSpec((T, H), lambda i: (0, 0)),
            pl.BlockSpec((T, 1), lambda i: (0, 0)),
        ],
        out_specs=pl.BlockSpec((T, L_SEQ, bt), lambda i: (0, 0, i)),
        out_shape=jax.ShapeDtypeStruct((T, L_SEQ, B), jnp.float32),
        scratch_shapes=[pltpu.VMEM((L_SEQ * bt, H), jnp.float32)],
    )(x, wih, whh, bias, wfc, bfc)


def kernel(sentences, labels, emb_table, W_ih, W_hh, b_ih, b_hh, W_fc, b_fc):
    B, L = sentences.shape
    n_rows = B * L
    idx = jnp.swapaxes(sentences, 0, 1).reshape(-1).astype(jnp.int32)
    idx3 = idx.reshape(_NW, -1, _CHUNK)
    x = _sc_gather(emb_table, idx3, n_rows).reshape(L, B, D)

    wih = W_ih.T
    whh = W_hh.T
    bias = (b_ih + b_hh).reshape(1, 4 * H)
    bfc = b_fc.reshape(T, 1)

    out_t = _lstm_fc(x, wih, whh, bias, W_fc, bfc)
    return jnp.transpose(out_t, (2, 1, 0))


# speedup vs baseline: 1.0435x; 1.0071x over previous
import functools

import jax
import jax.numpy as jnp
from jax import lax
from jax.experimental import pallas as pl
from jax.experimental.pallas import tpu as pltpu
from jax.experimental.pallas import tpu_sc as plsc

D = 32
H = 128
T = 9
L_SEQ = 50
P = 4

_NC = 2
_NS = 16
_NW = _NC * _NS
_CHUNK = 128


def _sc_gather(table, idx3, n_rows):
    n_chunks = idx3.shape[1]
    mesh = plsc.VectorSubcoreMesh(core_axis_name="c", subcore_axis_name="s")

    @functools.partial(
        pl.kernel,
        mesh=mesh,
        out_type=jax.ShapeDtypeStruct((n_rows, D), jnp.float32),
        compiler_params=pltpu.CompilerParams(use_tc_tiling_on_sc=False),
        scratch_types=[
            pltpu.VMEM((n_chunks, _CHUNK), jnp.int32),
            pltpu.VMEM((_CHUNK, D), jnp.float32),
            pltpu.SemaphoreType.DMA,
        ],
    )
    def k(table_hbm, idx_hbm, out_hbm, idx_v, rows_v, sem):
        wid = lax.axis_index("s") * _NC + lax.axis_index("c")
        pltpu.sync_copy(idx_hbm.at[wid], idx_v)

        def body(j, carry):
            pltpu.async_copy(table_hbm.at[idx_v.at[j]], rows_v, sem).wait()
            base = (wid * n_chunks + j) * _CHUNK
            pltpu.sync_copy(rows_v, out_hbm.at[pl.ds(base, _CHUNK)])
            return carry

        lax.fori_loop(0, n_chunks, body, 0)

    return k(table, idx3)


def _sigmoid(x):
    return 0.5 * jnp.tanh(0.5 * x) + 0.5


def _lstm_body(x_ref, w_ref, b_ref, wfc_ref, bfc_ref, out_ref, hs_ref):
    w = w_ref[...]
    b = b_ref[...]
    bp = x_ref.shape[1]
    HP = P * H

    def step(t, carry):
        h, c = carry
        inp = jnp.concatenate([x_ref[t], h], axis=1)
        gates = jnp.dot(inp, w, preferred_element_type=jnp.float32) + b
        ig = _sigmoid(gates[:, 0:HP])
        fg = _sigmoid(gates[:, HP:2 * HP])
        gg = jnp.tanh(gates[:, 2 * HP:3 * HP])
        og = _sigmoid(gates[:, 3 * HP:4 * HP])
        c = fg * c + ig * gg
        h = og * jnp.tanh(c)
        hs_ref[pl.ds(t * bp, bp), :] = h
        return (h, c)

    init = (jnp.zeros((bp, HP), jnp.float32), jnp.zeros((bp, HP), jnp.float32))
    lax.fori_loop(0, L_SEQ, step, init)

    wfc9 = wfc_ref[...]
    bfc9 = bfc_ref[...]
    for s in range(P):
        h_s = hs_ref[:, s * H:(s + 1) * H]
        lT = jax.lax.dot_general(wfc9, h_s, (((1,), (1,)), ((), ())),
                                 preferred_element_type=jnp.float32) + bfc9
        m = jnp.max(lT, axis=0, keepdims=True)
        e = jnp.exp(lT - m)
        lse = m + jnp.log(jnp.sum(e, axis=0, keepdims=True))
        out_ref[:, :, s * bp:(s + 1) * bp] = (lT - lse).reshape(T, L_SEQ, bp)


def _lstm_fc(x, w, bias, wfc, bfc, bp=128):
    BP = x.shape[1]
    return pl.pallas_call(
        _lstm_body,
        grid=(BP // bp,),
        in_specs=[
            pl.BlockSpec((L_SEQ, bp, P * D), lambda i: (0, i, 0)),
            pl.BlockSpec((P * (D + H), P * 4 * H), lambda i: (0, 0)),
            pl.BlockSpec((1, P * 4 * H), lambda i: (0, 0)),
            pl.BlockSpec((T, H), lambda i: (0, 0)),
            pl.BlockSpec((T, 1), lambda i: (0, 0)),
        ],
        out_specs=pl.BlockSpec((T, L_SEQ, P * bp), lambda i: (0, 0, i)),
        out_shape=jax.ShapeDtypeStruct((T, L_SEQ, P * BP), jnp.float32),
        scratch_shapes=[pltpu.VMEM((L_SEQ * bp, P * H), jnp.float32)],
    )(x, w, bias, wfc, bfc)


def _pack_w(w, eye):
    k = w.shape[0]
    wg = w.reshape(k, 4, H)
    m = jnp.einsum('st,kgh->skgth', eye, wg)
    return m.reshape(P * k, P * 4 * H)


def kernel(sentences, labels, emb_table, W_ih, W_hh, b_ih, b_hh, W_fc, b_fc):
    B, L = sentences.shape
    n_rows = B * L
    BP = B // P
    bp = 128
    sp = sentences.reshape(B // (P * bp), P, bp, L)
    idx = jnp.transpose(sp, (3, 0, 2, 1)).reshape(-1).astype(jnp.int32)
    idx3 = idx.reshape(_NW, -1, _CHUNK)
    x = _sc_gather(emb_table, idx3, n_rows).reshape(L, BP, P * D)

    eye = jnp.eye(P, dtype=jnp.float32)
    wih = _pack_w(W_ih.T, eye)
    whh = _pack_w(W_hh.T, eye)
    w = jnp.concatenate([wih, whh], axis=0)
    bg = (b_ih + b_hh).reshape(4, H)
    bias = jnp.broadcast_to(bg[:, None, :], (4, P, H)).reshape(1, P * 4 * H)
    bfc = b_fc.reshape(T, 1)

    out_t = _lstm_fc(x, w, bias, W_fc, bfc, bp=bp)
    return jnp.transpose(out_t, (2, 1, 0))


# speedup vs baseline: 1.0723x; 1.0277x over previous
import functools

import jax
import jax.numpy as jnp
from jax import lax
from jax.experimental import pallas as pl
from jax.experimental.pallas import tpu as pltpu
from jax.experimental.pallas import tpu_sc as plsc

D = 32
H = 128
T = 9
L_SEQ = 50
P = 4

_NC = 2
_NS = 16
_NW = _NC * _NS
_CHUNK = 128


def _sc_gather(table, idx3, n_rows):
    n_chunks = idx3.shape[1]
    mesh = plsc.VectorSubcoreMesh(core_axis_name="c", subcore_axis_name="s")

    @functools.partial(
        pl.kernel,
        mesh=mesh,
        out_type=jax.ShapeDtypeStruct((n_rows, D), jnp.float32),
        compiler_params=pltpu.CompilerParams(use_tc_tiling_on_sc=False),
        scratch_types=[
            pltpu.VMEM((n_chunks, _CHUNK), jnp.int32),
            pltpu.VMEM((_CHUNK, D), jnp.float32),
            pltpu.SemaphoreType.DMA,
        ],
    )
    def k(table_hbm, idx_hbm, out_hbm, idx_v, rows_v, sem):
        wid = lax.axis_index("s") * _NC + lax.axis_index("c")
        pltpu.sync_copy(idx_hbm.at[wid], idx_v)

        def body(j, carry):
            pltpu.async_copy(table_hbm.at[idx_v.at[j]], rows_v, sem).wait()
            base = (wid * n_chunks + j) * _CHUNK
            pltpu.sync_copy(rows_v, out_hbm.at[pl.ds(base, _CHUNK)])
            return carry

        lax.fori_loop(0, n_chunks, body, 0)

    return k(table, idx3)


def _sigmoid(x):
    return 0.5 * jnp.tanh(0.5 * x) + 0.5


def _lstm_body(x_ref, w_ref, b_ref, wfc_ref, bfc_ref, out_ref, hs_ref):
    w = w_ref[...]
    b = b_ref[...]
    bp = x_ref.shape[1]
    HP = P * H

    def step(t, carry):
        h, c = carry
        inp = jnp.concatenate([x_ref[t], h], axis=1)
        gates = jnp.dot(inp, w, preferred_element_type=jnp.float32) + b
        ig = _sigmoid(gates[:, 0:HP])
        fg = _sigmoid(gates[:, HP:2 * HP])
        gg = jnp.tanh(gates[:, 2 * HP:3 * HP])
        og = _sigmoid(gates[:, 3 * HP:4 * HP])
        c = fg * c + ig * gg
        h = og * jnp.tanh(c)
        hs_ref[pl.ds(t * bp, bp), :] = h
        return (h, c)

    init = (jnp.zeros((bp, HP), jnp.float32), jnp.zeros((bp, HP), jnp.float32))
    lax.fori_loop(0, L_SEQ, step, init)

    wfc9 = wfc_ref[...]
    bfc9 = bfc_ref[...]
    for s in range(P):
        h_s = hs_ref[:, s * H:(s + 1) * H]
        lT = jax.lax.dot_general(wfc9, h_s, (((1,), (1,)), ((), ())),
                                 preferred_element_type=jnp.float32) + bfc9
        m = jnp.max(lT, axis=0, keepdims=True)
        e = jnp.exp(lT - m)
        lse = m + jnp.log(jnp.sum(e, axis=0, keepdims=True))
        out_ref[:, :, s * bp:(s + 1) * bp] = (lT - lse).reshape(T, L_SEQ, bp)


def _lstm_fc(x, w, bias, wfc, bfc, bp=128):
    BP = x.shape[1]
    return pl.pallas_call(
        _lstm_body,
        grid=(BP // bp,),
        in_specs=[
            pl.BlockSpec((L_SEQ, bp, P * D), lambda i: (0, i, 0)),
            pl.BlockSpec((P * (D + H), P * 4 * H), lambda i: (0, 0)),
            pl.BlockSpec((1, P * 4 * H), lambda i: (0, 0)),
            pl.BlockSpec((T, H), lambda i: (0, 0)),
            pl.BlockSpec((T, 1), lambda i: (0, 0)),
        ],
        out_specs=pl.BlockSpec((T, L_SEQ, P * bp), lambda i: (0, 0, i)),
        out_shape=jax.ShapeDtypeStruct((T, L_SEQ, P * BP), jnp.float32),
        scratch_shapes=[pltpu.VMEM((L_SEQ * bp, P * H), jnp.float32)],
    )(x, w, bias, wfc, bfc)


def _pack_w(w, eye):
    k = w.shape[0]
    wg = w.reshape(k, 4, H)
    m = jnp.einsum('st,kgh->skgth', eye, wg)
    return m.reshape(P * k, P * 4 * H)


def kernel(sentences, labels, emb_table, W_ih, W_hh, b_ih, b_hh, W_fc, b_fc):
    B, L = sentences.shape
    n_rows = B * L
    BP = B // P
    bp = 256
    sp = sentences.reshape(B // (P * bp), P, bp, L)
    idx = jnp.transpose(sp, (3, 0, 2, 1)).reshape(-1).astype(jnp.int32)
    idx3 = idx.reshape(_NW, -1, _CHUNK)
    x = _sc_gather(emb_table, idx3, n_rows).reshape(L, BP, P * D)

    eye = jnp.eye(P, dtype=jnp.float32)
    wih = _pack_w(W_ih.T, eye)
    whh = _pack_w(W_hh.T, eye)
    w = jnp.concatenate([wih, whh], axis=0)
    bg = (b_ih + b_hh).reshape(4, H)
    bias = jnp.broadcast_to(bg[:, None, :], (4, P, H)).reshape(1, P * 4 * H)
    bfc = b_fc.reshape(T, 1)

    out_t = _lstm_fc(x, w, bias, W_fc, bfc, bp=bp)
    return jnp.transpose(out_t, (2, 1, 0))


# speedup vs baseline: 1.1208x; 1.0452x over previous
import functools

import jax
import jax.numpy as jnp
from jax import lax
from jax.experimental import pallas as pl
from jax.experimental.pallas import tpu as pltpu
from jax.experimental.pallas import tpu_sc as plsc

D = 32
H = 128
T = 9
L_SEQ = 50
P = 4

_NC = 2
_NS = 16
_NW = _NC * _NS
_CHUNK = 128


def _sc_gather(table, idx3, n_rows):
    n_chunks = idx3.shape[1]
    mesh = plsc.VectorSubcoreMesh(core_axis_name="c", subcore_axis_name="s")

    @functools.partial(
        pl.kernel,
        mesh=mesh,
        out_type=jax.ShapeDtypeStruct((n_rows, D), jnp.float32),
        compiler_params=pltpu.CompilerParams(use_tc_tiling_on_sc=False),
        scratch_types=[
            pltpu.VMEM((n_chunks, _CHUNK), jnp.int32),
            pltpu.VMEM((_CHUNK, D), jnp.float32),
            pltpu.SemaphoreType.DMA,
        ],
    )
    def k(table_hbm, idx_hbm, out_hbm, idx_v, rows_v, sem):
        wid = lax.axis_index("s") * _NC + lax.axis_index("c")
        pltpu.sync_copy(idx_hbm.at[wid], idx_v)

        def body(j, carry):
            pltpu.async_copy(table_hbm.at[idx_v.at[j]], rows_v, sem).wait()
            base = (wid * n_chunks + j) * _CHUNK
            pltpu.sync_copy(rows_v, out_hbm.at[pl.ds(base, _CHUNK)])
            return carry

        lax.fori_loop(0, n_chunks, body, 0)

    return k(table, idx3)


def _sigmoid(x):
    return 0.5 * jnp.tanh(0.5 * x) + 0.5


def _lstm_body(x_ref, w_ref, b_ref, wfc_ref, bfc_ref, out_ref, hs_ref):
    w = w_ref[...]
    b = b_ref[...]
    bp = x_ref.shape[1]
    HP = P * H

    def step(t, carry):
        h, c = carry
        inp = jnp.concatenate([x_ref[t], h], axis=1)
        gates = jnp.dot(inp, w, preferred_element_type=jnp.float32) + b
        ig = _sigmoid(gates[:, 0:HP])
        fg = _sigmoid(gates[:, HP:2 * HP])
        gg = jnp.tanh(gates[:, 2 * HP:3 * HP])
        og = _sigmoid(gates[:, 3 * HP:4 * HP])
        c = fg * c + ig * gg
        h = og * jnp.tanh(c)
        hs_ref[pl.ds(t * bp, bp), :] = h
        return (h, c)

    init = (jnp.zeros((bp, HP), jnp.float32), jnp.zeros((bp, HP), jnp.float32))
    lax.fori_loop(0, L_SEQ, step, init, unroll=2)

    wfc9 = wfc_ref[...]
    bfc9 = bfc_ref[...]
    for s in range(P):
        h_s = hs_ref[:, s * H:(s + 1) * H]
        lT = jax.lax.dot_general(wfc9, h_s, (((1,), (1,)), ((), ())),
                                 preferred_element_type=jnp.float32) + bfc9
        m = jnp.max(lT, axis=0, keepdims=True)
        e = jnp.exp(lT - m)
        lse = m + jnp.log(jnp.sum(e, axis=0, keepdims=True))
        out_ref[:, :, s * bp:(s + 1) * bp] = (lT - lse).reshape(T, L_SEQ, bp)


def _lstm_fc(x, w, bias, wfc, bfc, bp=128):
    BP = x.shape[1]
    return pl.pallas_call(
        _lstm_body,
        grid=(BP // bp,),
        in_specs=[
            pl.BlockSpec((L_SEQ, bp, P * D), lambda i: (0, i, 0)),
            pl.BlockSpec((P * (D + H), P * 4 * H), lambda i: (0, 0)),
            pl.BlockSpec((1, P * 4 * H), lambda i: (0, 0)),
            pl.BlockSpec((T, H), lambda i: (0, 0)),
            pl.BlockSpec((T, 1), lambda i: (0, 0)),
        ],
        out_specs=pl.BlockSpec((T, L_SEQ, P * bp), lambda i: (0, 0, i)),
        out_shape=jax.ShapeDtypeStruct((T, L_SEQ, P * BP), jnp.float32),
        scratch_shapes=[pltpu.VMEM((L_SEQ * bp, P * H), jnp.float32)],
    )(x, w, bias, wfc, bfc)


def _pack_w(w, eye):
    k = w.shape[0]
    wg = w.reshape(k, 4, H)
    m = jnp.einsum('st,kgh->skgth', eye, wg)
    return m.reshape(P * k, P * 4 * H)


def kernel(sentences, labels, emb_table, W_ih, W_hh, b_ih, b_hh, W_fc, b_fc):
    B, L = sentences.shape
    n_rows = B * L
    BP = B // P
    bp = 256
    sp = sentences.reshape(B // (P * bp), P, bp, L)
    idx = jnp.transpose(sp, (3, 0, 2, 1)).reshape(-1).astype(jnp.int32)
    idx3 = idx.reshape(_NW, -1, _CHUNK)
    x = _sc_gather(emb_table, idx3, n_rows).reshape(L, BP, P * D)

    eye = jnp.eye(P, dtype=jnp.float32)
    wih = _pack_w(W_ih.T, eye)
    whh = _pack_w(W_hh.T, eye)
    w = jnp.concatenate([wih, whh], axis=0)
    bg = (b_ih + b_hh).reshape(4, H)
    bias = jnp.broadcast_to(bg[:, None, :], (4, P, H)).reshape(1, P * 4 * H)
    bfc = b_fc.reshape(T, 1)

    out_t = _lstm_fc(x, w, bias, W_fc, bfc, bp=bp)
    return jnp.transpose(out_t, (2, 1, 0))


# speedup vs baseline: 1.1478x; 1.0241x over previous
import functools

import jax
import jax.numpy as jnp
from jax import lax
from jax.experimental import pallas as pl
from jax.experimental.pallas import tpu as pltpu
from jax.experimental.pallas import tpu_sc as plsc

D = 32
H = 128
T = 9
L_SEQ = 50
P = 4

_NC = 2
_NS = 16
_NW = _NC * _NS
_CHUNK = 128


def _sc_gather(table, idx3, n_rows):
    n_chunks = idx3.shape[1]
    mesh = plsc.VectorSubcoreMesh(core_axis_name="c", subcore_axis_name="s")

    @functools.partial(
        pl.kernel,
        mesh=mesh,
        out_type=jax.ShapeDtypeStruct((n_rows, D), jnp.float32),
        compiler_params=pltpu.CompilerParams(use_tc_tiling_on_sc=False),
        scratch_types=[
            pltpu.VMEM((n_chunks, _CHUNK), jnp.int32),
            pltpu.VMEM((_CHUNK, D), jnp.float32),
            pltpu.SemaphoreType.DMA,
        ],
    )
    def k(table_hbm, idx_hbm, out_hbm, idx_v, rows_v, sem):
        wid = lax.axis_index("s") * _NC + lax.axis_index("c")
        pltpu.sync_copy(idx_hbm.at[wid], idx_v)

        def body(j, carry):
            pltpu.async_copy(table_hbm.at[idx_v.at[j]], rows_v, sem).wait()
            base = (wid * n_chunks + j) * _CHUNK
            pltpu.sync_copy(rows_v, out_hbm.at[pl.ds(base, _CHUNK)])
            return carry

        lax.fori_loop(0, n_chunks, body, 0)

    return k(table, idx3)


def _sigmoid(x):
    return 0.5 * jnp.tanh(0.5 * x) + 0.5


def _lstm_body(x_ref, w_ref, b_ref, wfc_ref, bfc_ref, out_ref, hs_ref):
    w = w_ref[...]
    b = b_ref[...]
    bp = x_ref.shape[1]
    HP = P * H

    def step(t, carry):
        h, c = carry
        inp = jnp.concatenate([x_ref[t], h], axis=1)
        gates = jnp.dot(inp, w, preferred_element_type=jnp.float32) + b
        ig = _sigmoid(gates[:, 0:HP])
        fg = _sigmoid(gates[:, HP:2 * HP])
        gg = jnp.tanh(gates[:, 2 * HP:3 * HP])
        og = _sigmoid(gates[:, 3 * HP:4 * HP])
        c = fg * c + ig * gg
        h = og * jnp.tanh(c)
        hs_ref[pl.ds(t * bp, bp), :] = h
        return (h, c)

    init = (jnp.zeros((bp, HP), jnp.float32), jnp.zeros((bp, HP), jnp.float32))
    lax.fori_loop(0, L_SEQ, step, init, unroll=5)

    wfc9 = wfc_ref[...]
    bfc9 = bfc_ref[...]
    for s in range(P):
        h_s = hs_ref[:, s * H:(s + 1) * H]
        lT = jax.lax.dot_general(wfc9, h_s, (((1,), (1,)), ((), ())),
                                 preferred_element_type=jnp.float32) + bfc9
        m = jnp.max(lT, axis=0, keepdims=True)
        e = jnp.exp(lT - m)
        lse = m + jnp.log(jnp.sum(e, axis=0, keepdims=True))
        out_ref[:, :, s * bp:(s + 1) * bp] = (lT - lse).reshape(T, L_SEQ, bp)


def _lstm_fc(x, w, bias, wfc, bfc, bp=128):
    BP = x.shape[1]
    return pl.pallas_call(
        _lstm_body,
        grid=(BP // bp,),
        in_specs=[
            pl.BlockSpec((L_SEQ, bp, P * D), lambda i: (0, i, 0)),
            pl.BlockSpec((P * (D + H), P * 4 * H), lambda i: (0, 0)),
            pl.BlockSpec((1, P * 4 * H), lambda i: (0, 0)),
            pl.BlockSpec((T, H), lambda i: (0, 0)),
            pl.BlockSpec((T, 1), lambda i: (0, 0)),
        ],
        out_specs=pl.BlockSpec((T, L_SEQ, P * bp), lambda i: (0, 0, i)),
        out_shape=jax.ShapeDtypeStruct((T, L_SEQ, P * BP), jnp.float32),
        scratch_shapes=[pltpu.VMEM((L_SEQ * bp, P * H), jnp.float32)],
    )(x, w, bias, wfc, bfc)


def _pack_w(w, eye):
    k = w.shape[0]
    wg = w.reshape(k, 4, H)
    m = jnp.einsum('st,kgh->skgth', eye, wg)
    return m.reshape(P * k, P * 4 * H)


def kernel(sentences, labels, emb_table, W_ih, W_hh, b_ih, b_hh, W_fc, b_fc):
    B, L = sentences.shape
    n_rows = B * L
    BP = B // P
    bp = 256
    sp = sentences.reshape(B // (P * bp), P, bp, L)
    idx = jnp.transpose(sp, (3, 0, 2, 1)).reshape(-1).astype(jnp.int32)
    idx3 = idx.reshape(_NW, -1, _CHUNK)
    x = _sc_gather(emb_table, idx3, n_rows).reshape(L, BP, P * D)

    eye = jnp.eye(P, dtype=jnp.float32)
    wih = _pack_w(W_ih.T, eye)
    whh = _pack_w(W_hh.T, eye)
    w = jnp.concatenate([wih, whh], axis=0)
    bg = (b_ih + b_hh).reshape(4, H)
    bias = jnp.broadcast_to(bg[:, None, :], (4, P, H)).reshape(1, P * 4 * H)
    bfc = b_fc.reshape(T, 1)

    out_t = _lstm_fc(x, w, bias, W_fc, bfc, bp=bp)
    return jnp.transpose(out_t, (2, 1, 0))


# speedup vs baseline: 1.1566x; 1.0077x over previous
import functools

import jax
import jax.numpy as jnp
from jax import lax
from jax.experimental import pallas as pl
from jax.experimental.pallas import tpu as pltpu
from jax.experimental.pallas import tpu_sc as plsc

D = 32
H = 128
T = 9
L_SEQ = 50
P = 4

_NC = 2
_NS = 16
_NW = _NC * _NS
_CHUNK = 128


def _sc_gather(table, idx3, n_rows):
    n_chunks = idx3.shape[1]
    mesh = plsc.VectorSubcoreMesh(core_axis_name="c", subcore_axis_name="s")

    @functools.partial(
        pl.kernel,
        mesh=mesh,
        out_type=jax.ShapeDtypeStruct((n_rows, D), jnp.float32),
        compiler_params=pltpu.CompilerParams(use_tc_tiling_on_sc=False),
        scratch_types=[
            pltpu.VMEM((n_chunks, _CHUNK), jnp.int32),
            pltpu.VMEM((_CHUNK, D), jnp.float32),
            pltpu.SemaphoreType.DMA,
        ],
    )
    def k(table_hbm, idx_hbm, out_hbm, idx_v, rows_v, sem):
        wid = lax.axis_index("s") * _NC + lax.axis_index("c")
        pltpu.sync_copy(idx_hbm.at[wid], idx_v)

        def body(j, carry):
            pltpu.async_copy(table_hbm.at[idx_v.at[j]], rows_v, sem).wait()
            base = (wid * n_chunks + j) * _CHUNK
            pltpu.sync_copy(rows_v, out_hbm.at[pl.ds(base, _CHUNK)])
            return carry

        lax.fori_loop(0, n_chunks, body, 0)

    return k(table, idx3)


def _sigmoid(x):
    return 0.5 * jnp.tanh(0.5 * x) + 0.5


def _lstm_body(x_ref, w_ref, b_ref, wfc_ref, bfc_ref, out_ref, hs_ref):
    w = w_ref[...]
    b = b_ref[...]
    bp = x_ref.shape[1]
    HP = P * H

    def step(t, carry):
        h, c = carry
        inp = jnp.concatenate([x_ref[t], h], axis=1)
        gates = jnp.dot(inp, w, preferred_element_type=jnp.float32) + b
        ig = _sigmoid(gates[:, 0:HP])
        fg = _sigmoid(gates[:, HP:2 * HP])
        gg = jnp.tanh(gates[:, 2 * HP:3 * HP])
        og = _sigmoid(gates[:, 3 * HP:4 * HP])
        c = fg * c + ig * gg
        h = og * jnp.tanh(c)
        hs_ref[pl.ds(t * bp, bp), :] = h
        return (h, c)

    init = (jnp.zeros((bp, HP), jnp.float32), jnp.zeros((bp, HP), jnp.float32))
    lax.fori_loop(0, L_SEQ, step, init, unroll=10)

    wfc9 = wfc_ref[...]
    bfc9 = bfc_ref[...]
    for s in range(P):
        h_s = hs_ref[:, s * H:(s + 1) * H]
        lT = jax.lax.dot_general(wfc9, h_s, (((1,), (1,)), ((), ())),
                                 preferred_element_type=jnp.float32) + bfc9
        m = jnp.max(lT, axis=0, keepdims=True)
        e = jnp.exp(lT - m)
        lse = m + jnp.log(jnp.sum(e, axis=0, keepdims=True))
        out_ref[:, :, s * bp:(s + 1) * bp] = (lT - lse).reshape(T, L_SEQ, bp)


def _lstm_fc(x, w, bias, wfc, bfc, bp=128):
    BP = x.shape[1]
    return pl.pallas_call(
        _lstm_body,
        grid=(BP // bp,),
        in_specs=[
            pl.BlockSpec((L_SEQ, bp, P * D), lambda i: (0, i, 0)),
            pl.BlockSpec((P * (D + H), P * 4 * H), lambda i: (0, 0)),
            pl.BlockSpec((1, P * 4 * H), lambda i: (0, 0)),
            pl.BlockSpec((T, H), lambda i: (0, 0)),
            pl.BlockSpec((T, 1), lambda i: (0, 0)),
        ],
        out_specs=pl.BlockSpec((T, L_SEQ, P * bp), lambda i: (0, 0, i)),
        out_shape=jax.ShapeDtypeStruct((T, L_SEQ, P * BP), jnp.float32),
        scratch_shapes=[pltpu.VMEM((L_SEQ * bp, P * H), jnp.float32)],
    )(x, w, bias, wfc, bfc)


def _pack_w(w, eye):
    k = w.shape[0]
    wg = w.reshape(k, 4, H)
    m = jnp.einsum('st,kgh->skgth', eye, wg)
    return m.reshape(P * k, P * 4 * H)


def kernel(sentences, labels, emb_table, W_ih, W_hh, b_ih, b_hh, W_fc, b_fc):
    B, L = sentences.shape
    n_rows = B * L
    BP = B // P
    bp = 256
    sp = sentences.reshape(B // (P * bp), P, bp, L)
    idx = jnp.transpose(sp, (3, 0, 2, 1)).reshape(-1).astype(jnp.int32)
    idx3 = idx.reshape(_NW, -1, _CHUNK)
    x = _sc_gather(emb_table, idx3, n_rows).reshape(L, BP, P * D)

    eye = jnp.eye(P, dtype=jnp.float32)
    wih = _pack_w(W_ih.T, eye)
    whh = _pack_w(W_hh.T, eye)
    w = jnp.concatenate([wih, whh], axis=0)
    bg = (b_ih + b_hh).reshape(4, H)
    bias = jnp.broadcast_to(bg[:, None, :], (4, P, H)).reshape(1, P * 4 * H)
    bfc = b_fc.reshape(T, 1)

    out_t = _lstm_fc(x, w, bias, W_fc, bfc, bp=bp)
    return jnp.transpose(out_t, (2, 1, 0))
